# baseline (speedup 1.0000x reference)
import functools

import jax
import jax.numpy as jnp
from jax import lax
from jax.experimental import pallas as pl
from jax.experimental.pallas import tpu as pltpu
from jax.experimental.pallas import tpu_sc as plsc

B, N, M, CIN, COUT = 4, 1024, 4096, 256, 128
MB = 1024
MBF = 2048
NW = 32
RPW = (B * M) // NW
CH = 64
NCH = RPW // CH
EPS_BN = 1e-5
EPS_W = 1e-8


def _prep_h_kernel(x1_ref, wup_ref, g1_ref, b1_ref, ht_ref):
    wup = wup_ref[...]
    s = jnp.zeros((1, COUT), jnp.float32)
    ss = jnp.zeros((1, COUT), jnp.float32)
    for b in range(B):
        hb = lax.dot_general(x1_ref[b], wup, (((0,), (1,)), ((), ())),
                             preferred_element_type=jnp.float32)
        ht_ref[pl.ds(b * N, N)] = hb
        s = s + jnp.sum(hb, axis=0, keepdims=True)
        ss = ss + jnp.sum(hb * hb, axis=0, keepdims=True)
    cnt = float(B * N)
    mean = s / cnt
    var = ss / cnt - mean * mean
    scale = g1_ref[...].reshape(1, COUT) * lax.rsqrt(var + EPS_BN)
    shift = b1_ref[...].reshape(1, COUT) - mean * scale
    for b in range(B):
        ht_ref[pl.ds(b * N, N)] = jnp.maximum(
            ht_ref[pl.ds(b * N, N)] * scale + shift, 0.0)


def _prep_l_kernel(x2_ref, wlat_ref, g2_ref, b2_ref, l_ref):
    wlat = wlat_ref[...]
    s = jnp.zeros((COUT, 1), jnp.float32)
    ss = jnp.zeros((COUT, 1), jnp.float32)
    for b in range(B):
        lb = jnp.dot(wlat, x2_ref[b], preferred_element_type=jnp.float32)
        l_ref[b] = lb
        s = s + jnp.sum(lb, axis=1, keepdims=True)
        ss = ss + jnp.sum(lb * lb, axis=1, keepdims=True)
    cnt = float(B * M)
    mean = s / cnt
    var = ss / cnt - mean * mean
    scale = g2_ref[...].reshape(COUT, 1) * lax.rsqrt(var + EPS_BN)
    shift = b2_ref[...].reshape(COUT, 1) - mean * scale
    for b in range(B):
        l_ref[b] = jnp.maximum(l_ref[b] * scale + shift, 0.0)


def _knn_kernel(p2_ref, p1_ref, gidx_ref, wexp_ref):
    b = pl.program_id(0)
    p2b = p2_ref[0]
    p1b = p1_ref[0]

    sqd = (
        jnp.sum(p2b * p2b, axis=1, keepdims=True)
        + jnp.sum(p1b * p1b, axis=1, keepdims=True).reshape(1, N)
        - 2.0 * lax.dot_general(p2b, p1b, (((1,), (1,)), ((), ())),
                                preferred_element_type=jnp.float32)
    )

    lane_iota = lax.broadcasted_iota(jnp.int32, (MB, N), 1)
    dists = []
    idxs = []
    for _ in range(3):
        d = jnp.min(sqd, axis=1, keepdims=True)
        cand = jnp.where(sqd == d, lane_iota, N)
        i = jnp.min(cand, axis=1, keepdims=True)
        sqd = jnp.where(lane_iota == i, jnp.float32(3.4e38), sqd)
        dists.append(d)
        idxs.append(i)

    recips = [1.0 / (d + EPS_W) for d in dists]
    norm = recips[0] + recips[1] + recips[2]
    for k in range(3):
        gidx_ref[0, :, pl.ds(k, 1)] = idxs[k] + b * N
        wk = recips[k] / norm
        wexp_ref[0, :, pl.ds(16 * k, 16)] = jnp.broadcast_to(wk, (MB, 16))


def _sc_interp_body(ht_ref, gidx_ref, wexp_ref, out_ref,
                    idx_a, w_a, rows_a, out_a, sem_a,
                    idx_b, w_b, rows_b, out_b, sem_b):
    wid = lax.axis_index("s") * 2 + lax.axis_index("c")
    base = wid * RPW

    def issue(ci, idx_v, w_v, rows_v, sem):
        flat = base + ci * CH
        pltpu.sync_copy(gidx_ref.at[pl.ds(flat * 3, 3 * CH)], idx_v)
        pltpu.sync_copy(wexp_ref.at[pl.ds(flat, CH)], w_v)
        pltpu.async_copy(ht_ref.at[idx_v], rows_v, sem)

    def combine(ci, idx_v, w_v, rows_v, out_v, sem):
        pltpu.make_async_copy(ht_ref.at[idx_v], rows_v, sem).wait()

        def mstep(mg, carry):
            for u in range(8):
                m = mg * 8 + u
                w0 = w_v[m, pl.ds(0, 16)]
                w1 = w_v[m, pl.ds(16, 16)]
                w2 = w_v[m, pl.ds(32, 16)]
                for c in range(COUT // 16):
                    sl = pl.ds(c * 16, 16)
                    out_v[m, sl] = (rows_v[3 * m, sl] * w0
                                    + rows_v[3 * m + 1, sl] * w1
                                    + rows_v[3 * m + 2, sl] * w2)
            return carry

        lax.fori_loop(0, CH // 8, mstep, 0)
        flat = base + ci * CH
        pltpu.sync_copy(out_v, out_ref.at[pl.ds(flat, CH)])

    issue(0, idx_a, w_a, rows_a, sem_a)

    def step(pi, carry):
        ci = pi * 2
        issue(ci + 1, idx_b, w_b, rows_b, sem_b)
        combine(ci, idx_a, w_a, rows_a, out_a, sem_a)

        @pl.when(pi < NCH // 2 - 1)
        def _():
            issue(ci + 2, idx_a, w_a, rows_a, sem_a)

        combine(ci + 1, idx_b, w_b, rows_b, out_b, sem_b)
        return carry

    lax.fori_loop(0, NCH // 2, step, 0)


@functools.cache
def _get_sc_interp():
    buf = lambda: [
        pltpu.VMEM((3 * CH,), jnp.int32),
        pltpu.VMEM((CH, 48), jnp.float32),
        pltpu.VMEM((3 * CH, COUT), jnp.float32),
        pltpu.VMEM((CH, COUT), jnp.float32),
        pltpu.SemaphoreType.DMA,
    ]
    return pl.kernel(
        _sc_interp_body,
        out_type=jax.ShapeDtypeStruct((B * M, COUT), jnp.float32),
        mesh=plsc.VectorSubcoreMesh(core_axis_name="c", subcore_axis_name="s",
                                    num_cores=2, num_subcores=16),
        scratch_types=buf() + buf(),
    )


def _sc_interp(ht, gidx, wexp):
    return _get_sc_interp()(ht, gidx, wexp)


def _final_kernel(it_ref, l_ref, out_ref):
    a = it_ref[0]
    r = lax.broadcasted_iota(jnp.int32, (COUT, COUT), 0)
    c = lax.broadcasted_iota(jnp.int32, (COUT, COUT), 1)
    eye = jnp.where(r == c, 1.0, 0.0).astype(jnp.float32)
    at = lax.dot_general(eye, a, (((1,), (1,)), ((), ())),
                         preferred_element_type=jnp.float32)
    out_ref[0] = at + l_ref[0]


@jax.jit
def kernel(x1, p1, x2, p2, W_up, gamma1, beta1, W_lat, gamma2, beta2):
    ht = pl.pallas_call(
        _prep_h_kernel,
        out_shape=jax.ShapeDtypeStruct((B * N, COUT), jnp.float32),
    )(x1, W_up, gamma1, beta1)

    l = pl.pallas_call(
        _prep_l_kernel,
        out_shape=jax.ShapeDtypeStruct((B, COUT, M), jnp.float32),
    )(x2, W_lat, gamma2, beta2)

    gidx, wexp = pl.pallas_call(
        _knn_kernel,
        grid=(B, M // MB),
        in_specs=[
            pl.BlockSpec((1, MB, 3), lambda b, m: (b, m, 0)),
            pl.BlockSpec((1, N, 3), lambda b, m: (b, 0, 0)),
        ],
        out_specs=[
            pl.BlockSpec((1, MB, 3), lambda b, m: (b, m, 0)),
            pl.BlockSpec((1, MB, 48), lambda b, m: (b, m, 0)),
        ],
        out_shape=(
            jax.ShapeDtypeStruct((B, M, 3), jnp.int32),
            jax.ShapeDtypeStruct((B, M, 48), jnp.float32),
        ),
    )(p2, p1)

    interp = _sc_interp(ht, gidx.reshape(B * M * 3), wexp.reshape(B * M, 48))

    out = pl.pallas_call(
        _final_kernel,
        grid=(B, M // MBF),
        in_specs=[
            pl.BlockSpec((1, MBF, COUT), lambda b, m: (b, m, 0)),
            pl.BlockSpec((1, COUT, MBF), lambda b, m: (b, 0, m)),
        ],
        out_specs=pl.BlockSpec((1, COUT, MBF), lambda b, m: (b, 0, m)),
        out_shape=jax.ShapeDtypeStruct((B, COUT, M), jnp.float32),
    )(interp.reshape(B, M, COUT), l)

    return (out, p2)


# speedup vs baseline: 1.1028x; 1.1028x over previous
import functools

import jax
import jax.numpy as jnp
from jax import lax
from jax.experimental import pallas as pl
from jax.experimental.pallas import tpu as pltpu
from jax.experimental.pallas import tpu_sc as plsc

B, N, M, CIN, COUT = 4, 1024, 4096, 256, 128
MB = 1024
MBF = 2048
NW = 32
BH = 2
RH = BH * M
RPW = RH // NW
CH = 64
NCH = RPW // CH
EPS_BN = 1e-5
EPS_W = 1e-8


def _prep_h_kernel(x1_ref, wup_ref, g1_ref, b1_ref, ht_ref):
    wup = wup_ref[...]
    s = jnp.zeros((1, COUT), jnp.float32)
    ss = jnp.zeros((1, COUT), jnp.float32)
    for b in range(B):
        hb = lax.dot_general(x1_ref[b], wup, (((0,), (1,)), ((), ())),
                             preferred_element_type=jnp.float32)
        ht_ref[pl.ds(b * N, N)] = hb
        s = s + jnp.sum(hb, axis=0, keepdims=True)
        ss = ss + jnp.sum(hb * hb, axis=0, keepdims=True)
    cnt = float(B * N)
    mean = s / cnt
    var = ss / cnt - mean * mean
    scale = g1_ref[...].reshape(1, COUT) * lax.rsqrt(var + EPS_BN)
    shift = b1_ref[...].reshape(1, COUT) - mean * scale
    for b in range(B):
        ht_ref[pl.ds(b * N, N)] = jnp.maximum(
            ht_ref[pl.ds(b * N, N)] * scale + shift, 0.0)


def _prep_l_kernel(x2_ref, wlat_ref, g2_ref, b2_ref, l_ref):
    wlat = wlat_ref[...]
    s = jnp.zeros((COUT, 1), jnp.float32)
    ss = jnp.zeros((COUT, 1), jnp.float32)
    for b in range(B):
        lb = jnp.dot(wlat, x2_ref[b], preferred_element_type=jnp.float32)
        l_ref[b] = lb
        s = s + jnp.sum(lb, axis=1, keepdims=True)
        ss = ss + jnp.sum(lb * lb, axis=1, keepdims=True)
    cnt = float(B * M)
    mean = s / cnt
    var = ss / cnt - mean * mean
    scale = g2_ref[...].reshape(COUT, 1) * lax.rsqrt(var + EPS_BN)
    shift = b2_ref[...].reshape(COUT, 1) - mean * scale
    for b in range(B):
        l_ref[b] = jnp.maximum(l_ref[b] * scale + shift, 0.0)


def _knn_kernel(p2_ref, p1_ref, gidx_ref, wexp_ref, *, b0):
    b = b0 + pl.program_id(0)
    p2b = p2_ref[0]
    p1b = p1_ref[0]

    sqd = (
        jnp.sum(p2b * p2b, axis=1, keepdims=True)
        + jnp.sum(p1b * p1b, axis=1, keepdims=True).reshape(1, N)
        - 2.0 * lax.dot_general(p2b, p1b, (((1,), (1,)), ((), ())),
                                preferred_element_type=jnp.float32)
    )

    lane_iota = lax.broadcasted_iota(jnp.int32, (MB, N), 1)
    dists = []
    idxs = []
    for _ in range(3):
        d = jnp.min(sqd, axis=1, keepdims=True)
        cand = jnp.where(sqd == d, lane_iota, N)
        i = jnp.min(cand, axis=1, keepdims=True)
        sqd = jnp.where(lane_iota == i, jnp.float32(3.4e38), sqd)
        dists.append(d)
        idxs.append(i)

    recips = [1.0 / (d + EPS_W) for d in dists]
    norm = recips[0] + recips[1] + recips[2]
    for k in range(3):
        gidx_ref[:, pl.ds(k, 1)] = idxs[k] + b * N
        wk = recips[k] / norm
        wexp_ref[:, pl.ds(16 * k, 16)] = jnp.broadcast_to(wk, (MB, 16))


def _sc_interp_body(ht_ref, gidx_ref, wexp_ref, out_ref,
                    idx_a, w_a, rows_a, out_a, sem_a,
                    idx_b, w_b, rows_b, out_b, sem_b):
    wid = lax.axis_index("s") * 2 + lax.axis_index("c")
    base = wid * RPW

    def issue(ci, idx_v, w_v, rows_v, sem):
        flat = base + ci * CH
        pltpu.sync_copy(gidx_ref.at[pl.ds(flat * 3, 3 * CH)], idx_v)
        pltpu.sync_copy(wexp_ref.at[pl.ds(flat, CH)], w_v)
        pltpu.async_copy(ht_ref.at[idx_v], rows_v, sem)

    def combine(ci, idx_v, w_v, rows_v, out_v, sem):
        pltpu.make_async_copy(ht_ref.at[idx_v], rows_v, sem).wait()

        def mstep(mg, carry):
            for u in range(8):
                m = mg * 8 + u
                w0 = w_v[m, pl.ds(0, 16)]
                w1 = w_v[m, pl.ds(16, 16)]
                w2 = w_v[m, pl.ds(32, 16)]
                for c in range(COUT // 16):
                    sl = pl.ds(c * 16, 16)
                    out_v[m, sl] = (rows_v[3 * m, sl] * w0
                                    + rows_v[3 * m + 1, sl] * w1
                                    + rows_v[3 * m + 2, sl] * w2)
            return carry

        lax.fori_loop(0, CH // 8, mstep, 0)
        flat = base + ci * CH
        pltpu.sync_copy(out_v, out_ref.at[pl.ds(flat, CH)])

    issue(0, idx_a, w_a, rows_a, sem_a)

    def step(pi, carry):
        ci = pi * 2
        issue(ci + 1, idx_b, w_b, rows_b, sem_b)
        combine(ci, idx_a, w_a, rows_a, out_a, sem_a)

        @pl.when(pi < NCH // 2 - 1)
        def _():
            issue(ci + 2, idx_a, w_a, rows_a, sem_a)

        combine(ci + 1, idx_b, w_b, rows_b, out_b, sem_b)
        return carry

    lax.fori_loop(0, NCH // 2, step, 0)


@functools.cache
def _get_sc_interp():
    buf = lambda: [
        pltpu.VMEM((3 * CH,), jnp.int32),
        pltpu.VMEM((CH, 48), jnp.float32),
        pltpu.VMEM((3 * CH, COUT), jnp.float32),
        pltpu.VMEM((CH, COUT), jnp.float32),
        pltpu.SemaphoreType.DMA,
    ]
    return pl.kernel(
        _sc_interp_body,
        out_type=jax.ShapeDtypeStruct((RH, COUT), jnp.float32),
        mesh=plsc.VectorSubcoreMesh(core_axis_name="c", subcore_axis_name="s",
                                    num_cores=2, num_subcores=16),
        scratch_types=buf() + buf(),
    )


def _sc_interp(ht, gidx, wexp):
    return _get_sc_interp()(ht, gidx, wexp)


def _final_kernel(it_ref, l_ref, out_ref):
    a = it_ref[...]
    r = lax.broadcasted_iota(jnp.int32, (COUT, COUT), 0)
    c = lax.broadcasted_iota(jnp.int32, (COUT, COUT), 1)
    eye = jnp.where(r == c, 1.0, 0.0).astype(jnp.float32)
    at = lax.dot_general(eye, a, (((1,), (1,)), ((), ())),
                         preferred_element_type=jnp.float32)
    out_ref[0] = at + l_ref[0]


@jax.jit
def kernel(x1, p1, x2, p2, W_up, gamma1, beta1, W_lat, gamma2, beta2):
    ht = pl.pallas_call(
        _prep_h_kernel,
        out_shape=jax.ShapeDtypeStruct((B * N, COUT), jnp.float32),
    )(x1, W_up, gamma1, beta1)

    l = pl.pallas_call(
        _prep_l_kernel,
        out_shape=jax.ShapeDtypeStruct((B, COUT, M), jnp.float32),
    )(x2, W_lat, gamma2, beta2)

    halves = []
    for h in range(B // BH):
        gidx, wexp = pl.pallas_call(
            functools.partial(_knn_kernel, b0=h * BH),
            grid=(BH, M // MB),
            in_specs=[
                pl.BlockSpec((1, MB, 3), lambda b, m, h=h: (h * BH + b, m, 0)),
                pl.BlockSpec((1, N, 3), lambda b, m, h=h: (h * BH + b, 0, 0)),
            ],
            out_specs=[
                pl.BlockSpec((MB, 3), lambda b, m: (b * (M // MB) + m, 0)),
                pl.BlockSpec((MB, 48), lambda b, m: (b * (M // MB) + m, 0)),
            ],
            out_shape=(
                jax.ShapeDtypeStruct((RH, 3), jnp.int32),
                jax.ShapeDtypeStruct((RH, 48), jnp.float32),
            ),
        )(p2, p1)

        interp = _sc_interp(ht, gidx.reshape(RH * 3), wexp)

        out_h = pl.pallas_call(
            _final_kernel,
            grid=(BH, M // MBF),
            in_specs=[
                pl.BlockSpec((MBF, COUT),
                             lambda b, m: (b * (M // MBF) + m, 0)),
                pl.BlockSpec((1, COUT, MBF),
                             lambda b, m, h=h: (h * BH + b, 0, m)),
            ],
            out_specs=pl.BlockSpec((1, COUT, MBF), lambda b, m: (b, 0, m)),
            out_shape=jax.ShapeDtypeStruct((BH, COUT, M), jnp.float32),
        )(interp, l)
        halves.append(out_h)

    out = jnp.concatenate(halves, axis=0)
    return (out, p2)
